# baseline (speedup 1.0000x reference)
import functools

import numpy as np
import jax
import jax.numpy as jnp
from jax import lax
from jax.experimental import pallas as pl
from jax.experimental.pallas import tpu as pltpu, tpu_sc as plsc

H = 128
NDIM = 3
NM = 10000
NG = 32768
E = 320000
CHUNK = 128
NWORK = 32
CPW = 80
E_PAD = NWORK * CPW * CHUNK
RPC = 4096
ACC_R = RPC + 16
TRASH = RPC
_SQRT_HALF = 0.7071067811865476


def _dot(a, b):
    return jnp.dot(a, b, precision=lax.Precision.HIGHEST)


def _dot3(a, b):
    a_hi = a.astype(jnp.bfloat16)
    a_lo = (a - a_hi.astype(jnp.float32)).astype(jnp.bfloat16)
    b_hi = b.astype(jnp.bfloat16)
    b_lo = (b - b_hi.astype(jnp.float32)).astype(jnp.bfloat16)
    f = jnp.float32
    return (jnp.dot(a_hi, b_hi, preferred_element_type=f)
            + jnp.dot(a_lo, b_hi, preferred_element_type=f)
            + jnp.dot(a_hi, b_lo, preferred_element_type=f))


def _gelu(v):
    return 0.5 * v * (1.0 + lax.erf(v * _SQRT_HALF))


def _embed_consts():
    dim, ndim = H, NDIM
    ndim_padding = dim % ndim
    dim_per = (dim - ndim_padding) // ndim
    padding = ndim_padding + (dim_per % 2) * ndim
    eff = (dim - padding) // ndim
    omega = 1.0 / (10000.0 ** (np.arange(0, eff, 2, dtype=np.float64) / eff))
    A = np.zeros((8, dim), np.float32)
    msin = np.zeros((1, dim), np.float32)
    mcos = np.zeros((1, dim), np.float32)
    half = eff // 2
    for j in range(ndim):
        for p in range(eff):
            k = j * eff + p
            if p < half:
                A[j, k] = omega[p]
                msin[0, k] = 1.0
            else:
                A[j, k] = omega[p - half]
                mcos[0, k] = 1.0
    return jnp.asarray(A), jnp.asarray(msin), jnp.asarray(mcos)


def _node_body(x_ref, p_ref, w1, b1, w2, b2, w3, b3, a_ref, ms_ref, mc_ref, o_ref):
    h = _gelu(_dot(x_ref[...], w1[...]) + b1[...])
    h = _gelu(_dot(h, w2[...]) + b2[...])
    h = _dot(h, w3[...]) + b3[...]
    ph = _dot(p_ref[...], a_ref[...])
    o_ref[...] = h + jnp.sin(ph) * ms_ref[...] + jnp.cos(ph) * mc_ref[...]


def _node_call(x, pos8, w1, b1, w2, b2, w3, b3, A, ms, mc):
    blk = 2000
    full = lambda s: pl.BlockSpec(s, lambda i: (0, 0))
    return pl.pallas_call(
        _node_body,
        grid=(NM // blk,),
        in_specs=[
            pl.BlockSpec((blk, H), lambda i: (i, 0)),
            pl.BlockSpec((blk, 8), lambda i: (i, 0)),
            full((H, H)), full((1, H)), full((H, H)), full((1, H)),
            full((H, H)), full((1, H)),
            full((8, H)), full((1, H)), full((1, H)),
        ],
        out_specs=pl.BlockSpec((blk, H), lambda i: (i, 0)),
        out_shape=jax.ShapeDtypeStruct((NM, H), jnp.float32),
    )(x, pos8, w1, b1, w2, b2, w3, b3, A, ms, mc)


def _embed_body(p_ref, a_ref, ms_ref, mc_ref, o_ref):
    ph = _dot(p_ref[...], a_ref[...])
    o_ref[...] = jnp.sin(ph) * ms_ref[...] + jnp.cos(ph) * mc_ref[...]


def _embed_call(pos8, A, ms, mc):
    blk = 4096
    full = lambda s: pl.BlockSpec(s, lambda i: (0, 0))
    return pl.pallas_call(
        _embed_body,
        grid=(NG // blk,),
        in_specs=[
            pl.BlockSpec((blk, 8), lambda i: (i, 0)),
            full((8, H)), full((1, H)), full((1, H)),
        ],
        out_specs=pl.BlockSpec((blk, H), lambda i: (i, 0)),
        out_shape=jax.ShapeDtypeStruct((NG, H), jnp.float32),
    )(pos8, A, ms, mc)


@functools.lru_cache(maxsize=None)
def _gather_kernel_fn():
    return functools.partial(
        pl.kernel,
        out_type=(jax.ShapeDtypeStruct((E_PAD, H), jnp.float32),
                  jax.ShapeDtypeStruct((E_PAD, H), jnp.float32)),
        mesh=plsc.VectorSubcoreMesh(core_axis_name="c", subcore_axis_name="s"),
        scratch_types=[
            pltpu.VMEM((CPW * CHUNK,), jnp.int32),
            pltpu.VMEM((CPW * CHUNK,), jnp.int32),
            pltpu.VMEM((CHUNK, H), jnp.float32),
            pltpu.VMEM((CHUNK, H), jnp.float32),
            pltpu.VMEM((CHUNK, H), jnp.float32),
            pltpu.VMEM((CHUNK, H), jnp.float32),
            pltpu.SemaphoreType.DMA,
            pltpu.SemaphoreType.DMA,
            pltpu.SemaphoreType.DMA,
            pltpu.SemaphoreType.DMA,
        ],
    )(_gather_body)


def _gather_body(h_hbm, ge_hbm, midx_hbm, gidx_hbm, hm_out, geg_out,
                 midx_all, gidx_all, hrow0, hrow1, grow0, grow1,
                 gsem0, gsem1, ssem0, ssem1):
    wid = lax.axis_index("s") * 2 + lax.axis_index("c")
    base0 = wid * (CPW * CHUNK)
    pltpu.sync_copy(midx_hbm.at[pl.ds(base0, CPW * CHUNK)], midx_all)
    pltpu.sync_copy(gidx_hbm.at[pl.ds(base0, CPW * CHUNK)], gidx_all)
    hrow = [hrow0, hrow1]
    grow = [grow0, grow1]
    gsem = [gsem0, gsem1]
    ssem = [ssem0, ssem1]

    def step(g, carry):
        descs = []
        for b in range(2):
            c = g * 2 + b

            @pl.when(g > 0)
            def _drain():
                pltpu.make_async_copy(
                    hrow[b], hm_out.at[pl.ds(base0, CHUNK)], ssem[b]).wait()
                pltpu.make_async_copy(
                    grow[b], geg_out.at[pl.ds(base0, CHUNK)], ssem[b]).wait()

            d1 = pltpu.async_copy(
                h_hbm.at[midx_all.at[pl.ds(c * CHUNK, CHUNK)]], hrow[b], gsem[b])
            d2 = pltpu.async_copy(
                ge_hbm.at[gidx_all.at[pl.ds(c * CHUNK, CHUNK)]], grow[b], gsem[b])
            descs.append((d1, d2))
        for b in range(2):
            base = base0 + (g * 2 + b) * CHUNK
            d1, d2 = descs[b]
            d1.wait()
            d2.wait()
            pltpu.async_copy(hrow[b], hm_out.at[pl.ds(base, CHUNK)], ssem[b])
            pltpu.async_copy(grow[b], geg_out.at[pl.ds(base, CHUNK)], ssem[b])
        return carry

    lax.fori_loop(0, CPW // 2, step, 0)
    for b in range(2):
        pltpu.make_async_copy(
            hrow[b], hm_out.at[pl.ds(base0, CHUNK)], ssem[b]).wait()
        pltpu.make_async_copy(
            grow[b], geg_out.at[pl.ds(base0, CHUNK)], ssem[b]).wait()


def _msg_body(hm_ref, ge_ref, w1a, w1b, b1, w2, b2, w3, b3, o_ref):
    t = _dot3(hm_ref[...], w1a[...]) + _dot3(ge_ref[...], w1b[...]) + b1[...]
    t = _gelu(t)
    t = _gelu(_dot3(t, w2[...]) + b2[...])
    o_ref[...] = _dot3(t, w3[...]) + b3[...]


def _msg_call(hm, geg, w1a, w1b, b1, w2, b2, w3, b3):
    blk = 2048
    full = lambda s: pl.BlockSpec(s, lambda i: (0, 0))
    return pl.pallas_call(
        _msg_body,
        grid=(E_PAD // blk,),
        in_specs=[
            pl.BlockSpec((blk, H), lambda i: (i, 0)),
            pl.BlockSpec((blk, H), lambda i: (i, 0)),
            full((H, 2 * H)), full((H, 2 * H)), full((1, 2 * H)),
            full((2 * H, H)), full((1, H)),
            full((H, H)), full((1, H)),
        ],
        out_specs=pl.BlockSpec((blk, H), lambda i: (i, 0)),
        out_shape=jax.ShapeDtypeStruct((E_PAD, H), jnp.float32),
    )(hm, geg, w1a, w1b, b1, w2, b2, w3, b3)


@functools.lru_cache(maxsize=None)
def _seg_kernel_fn():
    return functools.partial(
        pl.kernel,
        out_type=(jax.ShapeDtypeStruct((NG, H), jnp.float32),
                  jax.ShapeDtypeStruct((NG, H), jnp.float32)),
        mesh=plsc.VectorSubcoreMesh(core_axis_name="c", subcore_axis_name="s"),
        scratch_types=[
            pltpu.VMEM((16,), jnp.int32),
            pltpu.VMEM((CHUNK,), jnp.int32),
            pltpu.VMEM((CHUNK,), jnp.int32),
            pltpu.VMEM((CHUNK, H), jnp.float32),
            pltpu.VMEM((CHUNK, H), jnp.float32),
            pltpu.VMEM((CHUNK, H), jnp.float32),
            pltpu.VMEM_SHARED((ACC_R, H), jnp.float32),
            pltpu.VMEM_SHARED((ACC_R, H), jnp.float32),
        ],
    )(_seg_body)


def _seg_body(m_hbm, sgidx_hbm, bounds_hbm, zero_hbm, ones_hbm,
              sums_out, cnt_out,
              bounds_v, idxr_v, idxb_v, rows_v, ones_v, st_v, acc, acc_cnt):
    cid = lax.axis_index("c")
    sid = lax.axis_index("s")
    pltpu.sync_copy(bounds_hbm, bounds_v)
    pltpu.sync_copy(ones_hbm, ones_v)
    lanes = lax.iota(jnp.int32, 16)
    bv = bounds_v[...]
    b = [bv[i] for i in range(9)]
    rpt = RPC // 16

    for half in range(NG // RPC // 2):
        ck = cid * (NG // RPC // 2) + half
        rowbase = ck * RPC
        pltpu.sync_copy(zero_hbm, st_v)
        for j in range(rpt // CHUNK):
            pltpu.sync_copy(st_v, acc.at[pl.ds(sid * rpt + j * CHUNK, CHUNK)])
            pltpu.sync_copy(st_v, acc_cnt.at[pl.ds(sid * rpt + j * CHUNK, CHUNK)])

        @pl.when(sid == 15)
        def _clear_trash():
            pltpu.sync_copy(st_v.at[pl.ds(0, ACC_R - RPC)],
                            acc.at[pl.ds(RPC, ACC_R - RPC)])
            pltpu.sync_copy(st_v.at[pl.ds(0, ACC_R - RPC)],
                            acc_cnt.at[pl.ds(RPC, ACC_R - RPC)])

        plsc.subcore_barrier()
        lo = jnp.where(cid == 0, b[half], b[NG // RPC // 2 + half])
        hi = jnp.where(cid == 0, b[half + 1], b[NG // RPC // 2 + half + 1])
        alo = (lo // 8) * 8
        per = (((hi - alo) + 15) // 16 + 7) // 8 * 8
        nc = (per + CHUNK - 1) // CHUNK
        s0 = alo + sid * per
        send = s0 + per

        def estep(i, carry):
            st = jnp.minimum(s0 + i * CHUNK, E_PAD - CHUNK)
            pltpu.sync_copy(sgidx_hbm.at[pl.ds(st, CHUNK)], idxr_v)
            pltpu.sync_copy(m_hbm.at[pl.ds(st, CHUNK)], rows_v)
            for j in range(CHUNK // 16):
                rel = idxr_v[pl.ds(j * 16, 16)] - rowbase
                pos = st + j * 16 + lanes
                ok = (rel >= 0) & (rel < RPC) & (pos < send)
                idxb_v[pl.ds(j * 16, 16)] = jnp.where(ok, rel, TRASH)
            pltpu.sync_copy(rows_v, acc.at[idxb_v], add=True)
            pltpu.sync_copy(ones_v, acc_cnt.at[idxb_v], add=True)
            return carry

        lax.fori_loop(0, nc, estep, 0)
        plsc.subcore_barrier()
        for j in range(rpt // CHUNK):
            src = sid * rpt + j * CHUNK
            dst = rowbase + sid * rpt + j * CHUNK
            pltpu.sync_copy(acc.at[pl.ds(src, CHUNK)], rows_v)
            pltpu.sync_copy(rows_v, sums_out.at[pl.ds(dst, CHUNK)])
            pltpu.sync_copy(acc_cnt.at[pl.ds(src, CHUNK)], st_v)
            pltpu.sync_copy(st_v, cnt_out.at[pl.ds(dst, CHUNK)])
        plsc.subcore_barrier()


def _div_body(s_ref, c_ref, o_ref):
    c = jnp.maximum(c_ref[...][:, :1], 1.0)
    o_ref[...] = s_ref[...] / c


def _div_call(sums, cnt):
    blk = 4096
    return pl.pallas_call(
        _div_body,
        grid=(NG // blk,),
        in_specs=[
            pl.BlockSpec((blk, H), lambda i: (i, 0)),
            pl.BlockSpec((blk, H), lambda i: (i, 0)),
        ],
        out_specs=pl.BlockSpec((blk, H), lambda i: (i, 0)),
        out_shape=jax.ShapeDtypeStruct((NG, H), jnp.float32),
    )(sums, cnt)


def kernel(x, mesh_pos, grid_pos, mesh_to_grid_edges,
           W_in1, b_in1, W_in2, b_in2, W_in3, b_in3,
           W_m1, b_m1, W_m2, b_m2, W_m3, b_m3):
    grid_idx = mesh_to_grid_edges[:, 0]
    mesh_idx = mesh_to_grid_edges[:, 1]
    pad = E_PAD - E
    midx_p = jnp.concatenate([mesh_idx, jnp.zeros((pad,), jnp.int32)])
    gidx_p = jnp.concatenate([grid_idx, jnp.zeros((pad,), jnp.int32)])
    sgidx_p = jnp.concatenate([grid_idx, jnp.full((pad,), 1 << 29, jnp.int32)])
    bounds = jnp.searchsorted(
        grid_idx, jnp.arange(0, NG + 1, RPC, dtype=jnp.int32), side="left"
    ).astype(jnp.int32)
    bounds = jnp.concatenate([bounds, jnp.zeros((16 - bounds.shape[0],), jnp.int32)])

    A, ms, mc = _embed_consts()
    mesh_pos8 = jnp.pad(mesh_pos, ((0, 0), (0, 8 - NDIM)))
    grid_pos8 = jnp.pad(grid_pos, ((0, 0), (0, 8 - NDIM)))
    r = lambda b: b.reshape(1, -1)

    h = _node_call(x, mesh_pos8, W_in1, r(b_in1), W_in2, r(b_in2),
                   W_in3, r(b_in3), A, ms, mc)
    ge_tab = _embed_call(grid_pos8, A, ms, mc)
    hm, geg = _gather_kernel_fn()(h, ge_tab, midx_p, gidx_p)
    m = _msg_call(hm, geg, W_m1[:H], W_m1[H:], r(b_m1),
                  W_m2, r(b_m2), W_m3, r(b_m3))
    sums, cnt = _seg_kernel_fn()(
        m, sgidx_p, bounds,
        jnp.zeros((CHUNK, H), jnp.float32),
        jnp.ones((CHUNK, H), jnp.float32),
    )
    out = _div_call(sums, cnt)
    return out.reshape(1, NG, H)


# speedup vs baseline: 1.0613x; 1.0613x over previous
import functools

import numpy as np
import jax
import jax.numpy as jnp
from jax import lax
from jax.experimental import pallas as pl
from jax.experimental.pallas import tpu as pltpu, tpu_sc as plsc

H = 128
NDIM = 3
NM = 10000
NG = 32768
E = 320000
CHUNK = 128
NWORK = 32
CPW = 79
E_PAD = NWORK * CPW * CHUNK
RPC = 4096
ACC_R = RPC + 16
TRASH = RPC
_SQRT_HALF = 0.7071067811865476


def _dot(a, b):
    return jnp.dot(a, b, precision=lax.Precision.HIGHEST)


def _dot3(a, b):
    a_hi = a.astype(jnp.bfloat16)
    a_lo = (a - a_hi.astype(jnp.float32)).astype(jnp.bfloat16)
    b_hi = b.astype(jnp.bfloat16)
    b_lo = (b - b_hi.astype(jnp.float32)).astype(jnp.bfloat16)
    f = jnp.float32
    return (jnp.dot(a_hi, b_hi, preferred_element_type=f)
            + jnp.dot(a_lo, b_hi, preferred_element_type=f)
            + jnp.dot(a_hi, b_lo, preferred_element_type=f))


def _gelu(v):
    return 0.5 * v * (1.0 + lax.erf(v * _SQRT_HALF))


def _embed_consts():
    dim, ndim = H, NDIM
    ndim_padding = dim % ndim
    dim_per = (dim - ndim_padding) // ndim
    padding = ndim_padding + (dim_per % 2) * ndim
    eff = (dim - padding) // ndim
    omega = 1.0 / (10000.0 ** (np.arange(0, eff, 2, dtype=np.float64) / eff))
    A = np.zeros((8, dim), np.float32)
    msin = np.zeros((1, dim), np.float32)
    mcos = np.zeros((1, dim), np.float32)
    half = eff // 2
    for j in range(ndim):
        for p in range(eff):
            k = j * eff + p
            if p < half:
                A[j, k] = omega[p]
                msin[0, k] = 1.0
            else:
                A[j, k] = omega[p - half]
                mcos[0, k] = 1.0
    return jnp.asarray(A), jnp.asarray(msin), jnp.asarray(mcos)


def _node_body(x_ref, p_ref, w1, b1, w2, b2, w3, b3, a_ref, ms_ref, mc_ref, o_ref):
    h = _gelu(_dot(x_ref[...], w1[...]) + b1[...])
    h = _gelu(_dot(h, w2[...]) + b2[...])
    h = _dot(h, w3[...]) + b3[...]
    ph = _dot(p_ref[...], a_ref[...])
    o_ref[...] = h + jnp.sin(ph) * ms_ref[...] + jnp.cos(ph) * mc_ref[...]


def _node_call(x, pos8, w1, b1, w2, b2, w3, b3, A, ms, mc):
    blk = 2000
    full = lambda s: pl.BlockSpec(s, lambda i: (0, 0))
    return pl.pallas_call(
        _node_body,
        grid=(NM // blk,),
        in_specs=[
            pl.BlockSpec((blk, H), lambda i: (i, 0)),
            pl.BlockSpec((blk, 8), lambda i: (i, 0)),
            full((H, H)), full((1, H)), full((H, H)), full((1, H)),
            full((H, H)), full((1, H)),
            full((8, H)), full((1, H)), full((1, H)),
        ],
        out_specs=pl.BlockSpec((blk, H), lambda i: (i, 0)),
        out_shape=jax.ShapeDtypeStruct((NM, H), jnp.float32),
    )(x, pos8, w1, b1, w2, b2, w3, b3, A, ms, mc)


def _embed_body(p_ref, a_ref, ms_ref, mc_ref, o_ref):
    ph = _dot(p_ref[...], a_ref[...])
    o_ref[...] = jnp.sin(ph) * ms_ref[...] + jnp.cos(ph) * mc_ref[...]


def _embed_call(pos8, A, ms, mc):
    blk = 4096
    full = lambda s: pl.BlockSpec(s, lambda i: (0, 0))
    return pl.pallas_call(
        _embed_body,
        grid=(NG // blk,),
        in_specs=[
            pl.BlockSpec((blk, 8), lambda i: (i, 0)),
            full((8, H)), full((1, H)), full((1, H)),
        ],
        out_specs=pl.BlockSpec((blk, H), lambda i: (i, 0)),
        out_shape=jax.ShapeDtypeStruct((NG, H), jnp.float32),
    )(pos8, A, ms, mc)


@functools.lru_cache(maxsize=None)
def _gather_kernel_fn():
    return functools.partial(
        pl.kernel,
        out_type=(jax.ShapeDtypeStruct((E_PAD, H), jnp.float32),
                  jax.ShapeDtypeStruct((E_PAD, H), jnp.float32)),
        mesh=plsc.VectorSubcoreMesh(core_axis_name="c", subcore_axis_name="s"),
        scratch_types=[
            pltpu.VMEM((CHUNK,), jnp.int32),
            pltpu.VMEM((CHUNK,), jnp.int32),
            pltpu.VMEM((CHUNK, H), jnp.float32),
            pltpu.VMEM((CHUNK, H), jnp.float32),
            pltpu.SemaphoreType.DMA,
            pltpu.SemaphoreType.DMA,
        ],
    )(_gather_body)


def _gather_body(h_hbm, ge_hbm, midx_hbm, gidx_hbm, hm_out, geg_out,
                 midx_v, gidx_v, hrow_v, grow_v, sem1, sem2):
    wid = lax.axis_index("s") * 2 + lax.axis_index("c")
    base0 = wid * (CPW * CHUNK)

    def step(i, carry):
        base = base0 + i * CHUNK
        pltpu.sync_copy(midx_hbm.at[pl.ds(base, CHUNK)], midx_v)
        pltpu.sync_copy(gidx_hbm.at[pl.ds(base, CHUNK)], gidx_v)
        c1 = pltpu.async_copy(h_hbm.at[midx_v], hrow_v, sem1)
        c2 = pltpu.async_copy(ge_hbm.at[gidx_v], grow_v, sem2)
        c1.wait()
        c2.wait()
        pltpu.sync_copy(hrow_v, hm_out.at[pl.ds(base, CHUNK)])
        pltpu.sync_copy(grow_v, geg_out.at[pl.ds(base, CHUNK)])
        return carry

    lax.fori_loop(0, CPW, step, 0)


def _msg_body(hm_ref, ge_ref, w1a, w1b, b1, w2, b2, w3, b3, o_ref):
    t = _dot3(hm_ref[...], w1a[...]) + _dot3(ge_ref[...], w1b[...]) + b1[...]
    t = _gelu(t)
    t = _gelu(_dot3(t, w2[...]) + b2[...])
    o_ref[...] = _dot3(t, w3[...]) + b3[...]


def _msg_call(hm, geg, w1a, w1b, b1, w2, b2, w3, b3):
    blk = 2048
    full = lambda s: pl.BlockSpec(s, lambda i: (0, 0))
    return pl.pallas_call(
        _msg_body,
        grid=(E_PAD // blk,),
        in_specs=[
            pl.BlockSpec((blk, H), lambda i: (i, 0)),
            pl.BlockSpec((blk, H), lambda i: (i, 0)),
            full((H, 2 * H)), full((H, 2 * H)), full((1, 2 * H)),
            full((2 * H, H)), full((1, H)),
            full((H, H)), full((1, H)),
        ],
        out_specs=pl.BlockSpec((blk, H), lambda i: (i, 0)),
        out_shape=jax.ShapeDtypeStruct((E_PAD, H), jnp.float32),
    )(hm, geg, w1a, w1b, b1, w2, b2, w3, b3)


@functools.lru_cache(maxsize=None)
def _seg_kernel_fn():
    return functools.partial(
        pl.kernel,
        out_type=(jax.ShapeDtypeStruct((NG, H), jnp.float32),
                  jax.ShapeDtypeStruct((NG, H), jnp.float32)),
        mesh=plsc.VectorSubcoreMesh(core_axis_name="c", subcore_axis_name="s"),
        scratch_types=[
            pltpu.VMEM((16,), jnp.int32),
            pltpu.VMEM((CHUNK,), jnp.int32),
            pltpu.VMEM((CHUNK,), jnp.int32),
            pltpu.VMEM((CHUNK, H), jnp.float32),
            pltpu.VMEM((CHUNK, H), jnp.float32),
            pltpu.VMEM((CHUNK, H), jnp.float32),
            pltpu.VMEM_SHARED((ACC_R, H), jnp.float32),
            pltpu.VMEM_SHARED((ACC_R, H), jnp.float32),
        ],
    )(_seg_body)


def _seg_body(m_hbm, sgidx_hbm, bounds_hbm, zero_hbm, ones_hbm,
              sums_out, cnt_out,
              bounds_v, idxr_v, idxb_v, rows_v, ones_v, st_v, acc, acc_cnt):
    cid = lax.axis_index("c")
    sid = lax.axis_index("s")
    pltpu.sync_copy(bounds_hbm, bounds_v)
    pltpu.sync_copy(ones_hbm, ones_v)
    lanes = lax.iota(jnp.int32, 16)
    bv = bounds_v[...]
    b = [bv[i] for i in range(9)]
    rpt = RPC // 16

    for half in range(NG // RPC // 2):
        ck = cid * (NG // RPC // 2) + half
        rowbase = ck * RPC
        pltpu.sync_copy(zero_hbm, st_v)
        for j in range(rpt // CHUNK):
            pltpu.sync_copy(st_v, acc.at[pl.ds(sid * rpt + j * CHUNK, CHUNK)])
            pltpu.sync_copy(st_v, acc_cnt.at[pl.ds(sid * rpt + j * CHUNK, CHUNK)])

        @pl.when(sid == 15)
        def _clear_trash():
            pltpu.sync_copy(st_v.at[pl.ds(0, ACC_R - RPC)],
                            acc.at[pl.ds(RPC, ACC_R - RPC)])
            pltpu.sync_copy(st_v.at[pl.ds(0, ACC_R - RPC)],
                            acc_cnt.at[pl.ds(RPC, ACC_R - RPC)])

        plsc.subcore_barrier()
        lo = jnp.where(cid == 0, b[half], b[NG // RPC // 2 + half])
        hi = jnp.where(cid == 0, b[half + 1], b[NG // RPC // 2 + half + 1])
        alo = (lo // 8) * 8
        per = (((hi - alo) + 15) // 16 + 7) // 8 * 8
        nc = (per + CHUNK - 1) // CHUNK
        s0 = alo + sid * per
        send = s0 + per

        def estep(i, carry):
            st = jnp.minimum(s0 + i * CHUNK, E_PAD - CHUNK)
            pltpu.sync_copy(sgidx_hbm.at[pl.ds(st, CHUNK)], idxr_v)
            pltpu.sync_copy(m_hbm.at[pl.ds(st, CHUNK)], rows_v)
            for j in range(CHUNK // 16):
                rel = idxr_v[pl.ds(j * 16, 16)] - rowbase
                pos = st + j * 16 + lanes
                ok = (rel >= 0) & (rel < RPC) & (pos < send)
                idxb_v[pl.ds(j * 16, 16)] = jnp.where(ok, rel, TRASH)
            pltpu.sync_copy(rows_v, acc.at[idxb_v], add=True)
            pltpu.sync_copy(ones_v, acc_cnt.at[idxb_v], add=True)
            return carry

        lax.fori_loop(0, nc, estep, 0)
        plsc.subcore_barrier()
        for j in range(rpt // CHUNK):
            src = sid * rpt + j * CHUNK
            dst = rowbase + sid * rpt + j * CHUNK
            pltpu.sync_copy(acc.at[pl.ds(src, CHUNK)], rows_v)
            pltpu.sync_copy(rows_v, sums_out.at[pl.ds(dst, CHUNK)])
            pltpu.sync_copy(acc_cnt.at[pl.ds(src, CHUNK)], st_v)
            pltpu.sync_copy(st_v, cnt_out.at[pl.ds(dst, CHUNK)])
        plsc.subcore_barrier()


def _div_body(s_ref, c_ref, o_ref):
    c = jnp.maximum(c_ref[...][:, :1], 1.0)
    o_ref[...] = s_ref[...] / c


def _div_call(sums, cnt):
    blk = 4096
    return pl.pallas_call(
        _div_body,
        grid=(NG // blk,),
        in_specs=[
            pl.BlockSpec((blk, H), lambda i: (i, 0)),
            pl.BlockSpec((blk, H), lambda i: (i, 0)),
        ],
        out_specs=pl.BlockSpec((blk, H), lambda i: (i, 0)),
        out_shape=jax.ShapeDtypeStruct((NG, H), jnp.float32),
    )(sums, cnt)


def kernel(x, mesh_pos, grid_pos, mesh_to_grid_edges,
           W_in1, b_in1, W_in2, b_in2, W_in3, b_in3,
           W_m1, b_m1, W_m2, b_m2, W_m3, b_m3):
    grid_idx = mesh_to_grid_edges[:, 0]
    mesh_idx = mesh_to_grid_edges[:, 1]
    pad = E_PAD - E
    midx_p = jnp.concatenate([mesh_idx, jnp.zeros((pad,), jnp.int32)])
    gidx_p = jnp.concatenate([grid_idx, jnp.zeros((pad,), jnp.int32)])
    sgidx_p = jnp.concatenate([grid_idx, jnp.full((pad,), 1 << 29, jnp.int32)])
    bounds = jnp.searchsorted(
        grid_idx, jnp.arange(0, NG + 1, RPC, dtype=jnp.int32), side="left"
    ).astype(jnp.int32)
    bounds = jnp.concatenate([bounds, jnp.zeros((16 - bounds.shape[0],), jnp.int32)])

    A, ms, mc = _embed_consts()
    mesh_pos8 = jnp.pad(mesh_pos, ((0, 0), (0, 8 - NDIM)))
    grid_pos8 = jnp.pad(grid_pos, ((0, 0), (0, 8 - NDIM)))
    r = lambda b: b.reshape(1, -1)

    h = _node_call(x, mesh_pos8, W_in1, r(b_in1), W_in2, r(b_in2),
                   W_in3, r(b_in3), A, ms, mc)
    ge_tab = _embed_call(grid_pos8, A, ms, mc)
    hm, geg = _gather_kernel_fn()(h, ge_tab, midx_p, gidx_p)
    m = _msg_call(hm, geg, W_m1[:H], W_m1[H:], r(b_m1),
                  W_m2, r(b_m2), W_m3, r(b_m3))
    sums, cnt = _seg_kernel_fn()(
        m, sgidx_p, bounds,
        jnp.zeros((CHUNK, H), jnp.float32),
        jnp.ones((CHUNK, H), jnp.float32),
    )
    out = _div_call(sums, cnt)
    return out.reshape(1, NG, H)


# speedup vs baseline: 1.0749x; 1.0128x over previous
import functools

import numpy as np
import jax
import jax.numpy as jnp
from jax import lax
from jax.experimental import pallas as pl
from jax.experimental.pallas import tpu as pltpu, tpu_sc as plsc

H = 128
NDIM = 3
NM = 10000
NG = 32768
E = 320000
CHUNK = 128
NWORK = 32
CPW = 79
E_PAD = NWORK * CPW * CHUNK
RPC = 4096
ACC_R = RPC + 16
TRASH = RPC
_SQRT_HALF = 0.7071067811865476


def _dot(a, b):
    return jnp.dot(a, b, precision=lax.Precision.HIGHEST)


def _dot3(a, b):
    a_hi = a.astype(jnp.bfloat16)
    a_lo = (a - a_hi.astype(jnp.float32)).astype(jnp.bfloat16)
    b_hi = b.astype(jnp.bfloat16)
    b_lo = (b - b_hi.astype(jnp.float32)).astype(jnp.bfloat16)
    f = jnp.float32
    return (jnp.dot(a_hi, b_hi, preferred_element_type=f)
            + jnp.dot(a_lo, b_hi, preferred_element_type=f)
            + jnp.dot(a_hi, b_lo, preferred_element_type=f))


def _gelu(v):
    return 0.5 * v * (1.0 + lax.erf(v * _SQRT_HALF))


def _embed_consts():
    dim, ndim = H, NDIM
    ndim_padding = dim % ndim
    dim_per = (dim - ndim_padding) // ndim
    padding = ndim_padding + (dim_per % 2) * ndim
    eff = (dim - padding) // ndim
    omega = 1.0 / (10000.0 ** (np.arange(0, eff, 2, dtype=np.float64) / eff))
    A = np.zeros((8, dim), np.float32)
    msin = np.zeros((1, dim), np.float32)
    mcos = np.zeros((1, dim), np.float32)
    half = eff // 2
    for j in range(ndim):
        for p in range(eff):
            k = j * eff + p
            if p < half:
                A[j, k] = omega[p]
                msin[0, k] = 1.0
            else:
                A[j, k] = omega[p - half]
                mcos[0, k] = 1.0
    return jnp.asarray(A), jnp.asarray(msin), jnp.asarray(mcos)


def _node_body(x_ref, p_ref, w1, b1, w2, b2, w3, b3, a_ref, ms_ref, mc_ref, o_ref):
    h = _gelu(_dot(x_ref[...], w1[...]) + b1[...])
    h = _gelu(_dot(h, w2[...]) + b2[...])
    h = _dot(h, w3[...]) + b3[...]
    ph = _dot(p_ref[...], a_ref[...])
    o_ref[...] = h + jnp.sin(ph) * ms_ref[...] + jnp.cos(ph) * mc_ref[...]


def _node_call(x, pos8, w1, b1, w2, b2, w3, b3, A, ms, mc):
    blk = 2000
    full = lambda s: pl.BlockSpec(s, lambda i: (0, 0))
    return pl.pallas_call(
        _node_body,
        grid=(NM // blk,),
        in_specs=[
            pl.BlockSpec((blk, H), lambda i: (i, 0)),
            pl.BlockSpec((blk, 8), lambda i: (i, 0)),
            full((H, H)), full((1, H)), full((H, H)), full((1, H)),
            full((H, H)), full((1, H)),
            full((8, H)), full((1, H)), full((1, H)),
        ],
        out_specs=pl.BlockSpec((blk, H), lambda i: (i, 0)),
        out_shape=jax.ShapeDtypeStruct((NM, H), jnp.float32),
    )(x, pos8, w1, b1, w2, b2, w3, b3, A, ms, mc)


def _embed_body(p_ref, a_ref, ms_ref, mc_ref, o_ref):
    ph = _dot(p_ref[...], a_ref[...])
    o_ref[...] = jnp.sin(ph) * ms_ref[...] + jnp.cos(ph) * mc_ref[...]


def _embed_call(pos8, A, ms, mc):
    blk = 4096
    full = lambda s: pl.BlockSpec(s, lambda i: (0, 0))
    return pl.pallas_call(
        _embed_body,
        grid=(NG // blk,),
        in_specs=[
            pl.BlockSpec((blk, 8), lambda i: (i, 0)),
            full((8, H)), full((1, H)), full((1, H)),
        ],
        out_specs=pl.BlockSpec((blk, H), lambda i: (i, 0)),
        out_shape=jax.ShapeDtypeStruct((NG, H), jnp.float32),
    )(pos8, A, ms, mc)


@functools.lru_cache(maxsize=None)
def _gather_kernel_fn():
    return functools.partial(
        pl.kernel,
        out_type=(jax.ShapeDtypeStruct((E_PAD, H), jnp.float32),
                  jax.ShapeDtypeStruct((E_PAD, H), jnp.float32)),
        mesh=plsc.VectorSubcoreMesh(core_axis_name="c", subcore_axis_name="s"),
        scratch_types=[
            pltpu.VMEM((CHUNK,), jnp.int32),
            pltpu.VMEM((CHUNK,), jnp.int32),
            pltpu.VMEM((CHUNK, H), jnp.float32),
            pltpu.VMEM((CHUNK, H), jnp.float32),
            pltpu.SemaphoreType.DMA,
            pltpu.SemaphoreType.DMA,
        ],
    )(_gather_body)


def _gather_body(h_hbm, ge_hbm, midx_hbm, gidx_hbm, hm_out, geg_out,
                 midx_v, gidx_v, hrow_v, grow_v, sem1, sem2):
    wid = lax.axis_index("s") * 2 + lax.axis_index("c")
    base0 = wid * (CPW * CHUNK)

    def step(i, carry):
        base = base0 + i * CHUNK
        pltpu.sync_copy(midx_hbm.at[pl.ds(base, CHUNK)], midx_v)
        pltpu.sync_copy(gidx_hbm.at[pl.ds(base, CHUNK)], gidx_v)
        c1 = pltpu.async_copy(h_hbm.at[midx_v], hrow_v, sem1)
        c2 = pltpu.async_copy(ge_hbm.at[gidx_v], grow_v, sem2)
        c1.wait()
        c2.wait()
        pltpu.sync_copy(hrow_v, hm_out.at[pl.ds(base, CHUNK)])
        pltpu.sync_copy(grow_v, geg_out.at[pl.ds(base, CHUNK)])
        return carry

    lax.fori_loop(0, CPW, step, 0)


def _msg_body(hm_ref, ge_ref, w1a, w1b, b1, w2, b2, w3, b3, o_ref):
    t = _dot3(hm_ref[...], w1a[...]) + _dot3(ge_ref[...], w1b[...]) + b1[...]
    t = _gelu(t)
    t = _gelu(_dot3(t, w2[...]) + b2[...])
    o_ref[...] = _dot3(t, w3[...]) + b3[...]


def _msg_call(hm, geg, w1a, w1b, b1, w2, b2, w3, b3):
    blk = 4096
    full = lambda s: pl.BlockSpec(s, lambda i: (0, 0))
    return pl.pallas_call(
        _msg_body,
        grid=(E_PAD // blk,),
        in_specs=[
            pl.BlockSpec((blk, H), lambda i: (i, 0)),
            pl.BlockSpec((blk, H), lambda i: (i, 0)),
            full((H, 2 * H)), full((H, 2 * H)), full((1, 2 * H)),
            full((2 * H, H)), full((1, H)),
            full((H, H)), full((1, H)),
        ],
        out_specs=pl.BlockSpec((blk, H), lambda i: (i, 0)),
        out_shape=jax.ShapeDtypeStruct((E_PAD, H), jnp.float32),
    )(hm, geg, w1a, w1b, b1, w2, b2, w3, b3)


@functools.lru_cache(maxsize=None)
def _seg_kernel_fn():
    return functools.partial(
        pl.kernel,
        out_type=(jax.ShapeDtypeStruct((NG, H), jnp.float32),
                  jax.ShapeDtypeStruct((NG, H), jnp.float32)),
        mesh=plsc.VectorSubcoreMesh(core_axis_name="c", subcore_axis_name="s"),
        scratch_types=[
            pltpu.VMEM((16,), jnp.int32),
            pltpu.VMEM((CHUNK,), jnp.int32),
            pltpu.VMEM((CHUNK,), jnp.int32),
            pltpu.VMEM((CHUNK, H), jnp.float32),
            pltpu.VMEM((CHUNK, H), jnp.float32),
            pltpu.VMEM((CHUNK, H), jnp.float32),
            pltpu.VMEM_SHARED((ACC_R, H), jnp.float32),
            pltpu.VMEM_SHARED((ACC_R, H), jnp.float32),
        ],
    )(_seg_body)


def _seg_body(m_hbm, sgidx_hbm, bounds_hbm, zero_hbm, ones_hbm,
              sums_out, cnt_out,
              bounds_v, idxr_v, idxb_v, rows_v, ones_v, st_v, acc, acc_cnt):
    cid = lax.axis_index("c")
    sid = lax.axis_index("s")
    pltpu.sync_copy(bounds_hbm, bounds_v)
    pltpu.sync_copy(ones_hbm, ones_v)
    lanes = lax.iota(jnp.int32, 16)
    bv = bounds_v[...]
    b = [bv[i] for i in range(9)]
    rpt = RPC // 16

    for half in range(NG // RPC // 2):
        ck = cid * (NG // RPC // 2) + half
        rowbase = ck * RPC
        pltpu.sync_copy(zero_hbm, st_v)
        for j in range(rpt // CHUNK):
            pltpu.sync_copy(st_v, acc.at[pl.ds(sid * rpt + j * CHUNK, CHUNK)])
            pltpu.sync_copy(st_v, acc_cnt.at[pl.ds(sid * rpt + j * CHUNK, CHUNK)])

        @pl.when(sid == 15)
        def _clear_trash():
            pltpu.sync_copy(st_v.at[pl.ds(0, ACC_R - RPC)],
                            acc.at[pl.ds(RPC, ACC_R - RPC)])
            pltpu.sync_copy(st_v.at[pl.ds(0, ACC_R - RPC)],
                            acc_cnt.at[pl.ds(RPC, ACC_R - RPC)])

        plsc.subcore_barrier()
        lo = jnp.where(cid == 0, b[half], b[NG // RPC // 2 + half])
        hi = jnp.where(cid == 0, b[half + 1], b[NG // RPC // 2 + half + 1])
        alo = (lo // 8) * 8
        per = (((hi - alo) + 15) // 16 + 7) // 8 * 8
        nc = (per + CHUNK - 1) // CHUNK
        s0 = alo + sid * per
        send = s0 + per

        def estep(i, carry):
            st = jnp.minimum(s0 + i * CHUNK, E_PAD - CHUNK)
            pltpu.sync_copy(sgidx_hbm.at[pl.ds(st, CHUNK)], idxr_v)
            pltpu.sync_copy(m_hbm.at[pl.ds(st, CHUNK)], rows_v)
            for j in range(CHUNK // 16):
                rel = idxr_v[pl.ds(j * 16, 16)] - rowbase
                pos = st + j * 16 + lanes
                ok = (rel >= 0) & (rel < RPC) & (pos < send)
                idxb_v[pl.ds(j * 16, 16)] = jnp.where(ok, rel, TRASH)
            pltpu.sync_copy(rows_v, acc.at[idxb_v], add=True)
            pltpu.sync_copy(ones_v, acc_cnt.at[idxb_v], add=True)
            return carry

        lax.fori_loop(0, nc, estep, 0)
        plsc.subcore_barrier()
        for j in range(rpt // CHUNK):
            src = sid * rpt + j * CHUNK
            dst = rowbase + sid * rpt + j * CHUNK
            pltpu.sync_copy(acc.at[pl.ds(src, CHUNK)], rows_v)
            pltpu.sync_copy(rows_v, sums_out.at[pl.ds(dst, CHUNK)])
            pltpu.sync_copy(acc_cnt.at[pl.ds(src, CHUNK)], st_v)
            pltpu.sync_copy(st_v, cnt_out.at[pl.ds(dst, CHUNK)])
        plsc.subcore_barrier()


def _div_body(s_ref, c_ref, o_ref):
    c = jnp.maximum(c_ref[...][:, :1], 1.0)
    o_ref[...] = s_ref[...] / c


def _div_call(sums, cnt):
    blk = 4096
    return pl.pallas_call(
        _div_body,
        grid=(NG // blk,),
        in_specs=[
            pl.BlockSpec((blk, H), lambda i: (i, 0)),
            pl.BlockSpec((blk, H), lambda i: (i, 0)),
        ],
        out_specs=pl.BlockSpec((blk, H), lambda i: (i, 0)),
        out_shape=jax.ShapeDtypeStruct((NG, H), jnp.float32),
    )(sums, cnt)


def kernel(x, mesh_pos, grid_pos, mesh_to_grid_edges,
           W_in1, b_in1, W_in2, b_in2, W_in3, b_in3,
           W_m1, b_m1, W_m2, b_m2, W_m3, b_m3):
    grid_idx = mesh_to_grid_edges[:, 0]
    mesh_idx = mesh_to_grid_edges[:, 1]
    pad = E_PAD - E
    midx_p = jnp.concatenate([mesh_idx, jnp.zeros((pad,), jnp.int32)])
    gidx_p = jnp.concatenate([grid_idx, jnp.zeros((pad,), jnp.int32)])
    sgidx_p = jnp.concatenate([grid_idx, jnp.full((pad,), 1 << 29, jnp.int32)])
    bounds = jnp.searchsorted(
        grid_idx, jnp.arange(0, NG + 1, RPC, dtype=jnp.int32), side="left"
    ).astype(jnp.int32)
    bounds = jnp.concatenate([bounds, jnp.zeros((16 - bounds.shape[0],), jnp.int32)])

    A, ms, mc = _embed_consts()
    mesh_pos8 = jnp.pad(mesh_pos, ((0, 0), (0, 8 - NDIM)))
    grid_pos8 = jnp.pad(grid_pos, ((0, 0), (0, 8 - NDIM)))
    r = lambda b: b.reshape(1, -1)

    h = _node_call(x, mesh_pos8, W_in1, r(b_in1), W_in2, r(b_in2),
                   W_in3, r(b_in3), A, ms, mc)
    ge_tab = _embed_call(grid_pos8, A, ms, mc)
    hm, geg = _gather_kernel_fn()(h, ge_tab, midx_p, gidx_p)
    m = _msg_call(hm, geg, W_m1[:H], W_m1[H:], r(b_m1),
                  W_m2, r(b_m2), W_m3, r(b_m3))
    sums, cnt = _seg_kernel_fn()(
        m, sgidx_p, bounds,
        jnp.zeros((CHUNK, H), jnp.float32),
        jnp.ones((CHUNK, H), jnp.float32),
    )
    out = _div_call(sums, cnt)
    return out.reshape(1, NG, H)
